# Initial kernel scaffold; baseline (speedup 1.0000x reference)
import jax
import jax.numpy as jnp
from jax.experimental import pallas as pl


def kernel(adj, x, normalized_embedding, w0, a_src0, a_dst0, b0, w1, a_src1, a_dst1, b1):
    raise NotImplementedError("write your pallas kernel here")


# speedup vs baseline: 2.6521x; 2.6521x over previous
import jax
import jax.numpy as jnp
from jax.experimental import pallas as pl
from jax.experimental.pallas import tpu as pltpu

_NEG = -1e9


def _leaky(x):
    return jnp.where(x >= 0, x, 0.2 * x)


def _gat_body(x_ref, e_ref, adj_ref, w0x_ref, w0e_ref, asrc0_ref, adst0_ref,
              b0_ref, w1_ref, asrc1_ref, adst1_ref, b1_ref, out_ref):
    n = x_ref.shape[1]
    heads, fo = asrc0_ref.shape

    xb = x_ref[0]
    eb = e_ref[0]
    h1 = (jnp.dot(xb, w0x_ref[...], preferred_element_type=jnp.float32)
          + jnp.dot(eb, w0e_ref[...], preferred_element_type=jnp.float32))
    adjm = adj_ref[0] != 0

    outs = []
    for h in range(heads):
        hh = h1[:, h * fo:(h + 1) * fo]
        av = asrc0_ref[h:h + 1, :]
        dv = adst0_ref[h:h + 1, :]
        asrc = jax.lax.dot_general(hh, av, (((1,), (1,)), ((), ())),
                                   preferred_element_type=jnp.float32)
        adst = jax.lax.dot_general(dv, hh, (((1,), (1,)), ((), ())),
                                   preferred_element_type=jnp.float32)
        lg = jnp.where(adjm, _leaky(asrc + adst), _NEG)
        m = jnp.max(lg, axis=1, keepdims=True)
        ex = jnp.exp(lg - m)
        p = ex / jnp.sum(ex, axis=1, keepdims=True)
        oh = jnp.dot(p, hh, preferred_element_type=jnp.float32) + b0_ref[...]
        outs.append(jnp.where(oh > 0, oh, jnp.exp(oh) - 1.0))

    o1 = jnp.concatenate(outs, axis=1)
    h2 = jnp.dot(o1, w1_ref[...], preferred_element_type=jnp.float32)

    hlast = h2[n - 1:n, :]
    s2 = jax.lax.dot_general(hlast, asrc1_ref[...], (((1,), (1,)), ((), ())),
                             preferred_element_type=jnp.float32)
    d2 = jax.lax.dot_general(adst1_ref[...], h2, (((1,), (1,)), ((), ())),
                             preferred_element_type=jnp.float32)
    lg2 = jnp.where(adjm[n - 1:n, :], _leaky(s2 + d2), _NEG)
    m2 = jnp.max(lg2, axis=1, keepdims=True)
    e2 = jnp.exp(lg2 - m2)
    p2 = e2 / jnp.sum(e2, axis=1, keepdims=True)
    o2 = jnp.dot(p2, h2, preferred_element_type=jnp.float32) + b1_ref[...]

    mm = jnp.max(o2, axis=1, keepdims=True)
    z = o2 - mm
    lse = jnp.log(jnp.sum(jnp.exp(z), axis=1, keepdims=True))
    out_ref[0] = z - lse


def kernel(adj, x, normalized_embedding, w0, a_src0, a_dst0, b0,
           w1, a_src1, a_dst1, b1):
    bs, n = adj.shape[:2]
    f_x = x.shape[2]
    f_emb = normalized_embedding.shape[2]
    h0, f_in0, f_out0 = w0.shape
    f_in1, f_out1 = w1.shape[1], w1.shape[2]

    emb = normalized_embedding.astype(jnp.float32)
    w0r = jnp.transpose(w0, (1, 0, 2)).reshape(f_in0, h0 * f_out0)
    w0x, w0e = w0r[:f_x], w0r[f_x:]
    asrc0m = a_src0[:, :, 0]
    adst0m = a_dst0[:, :, 0]
    w1m = w1[0]
    asrc1m = a_src1[0].reshape(1, f_out1)
    adst1m = a_dst1[0].reshape(1, f_out1)
    b0m = b0.reshape(1, f_out0)
    b1m = b1.reshape(1, f_out1)
    adj_i8 = adj.astype(jnp.int8)

    grid = (bs,)
    c0 = lambda b: (0, 0)
    specs = [
        pl.BlockSpec((1, n, n), lambda b: (b, 0, 0)),
        pl.BlockSpec((1, n, f_x), lambda b: (b, 0, 0)),
        pl.BlockSpec((1, n, f_emb), lambda b: (b, 0, 0)),
        pl.BlockSpec((f_x, h0 * f_out0), c0),
        pl.BlockSpec((f_emb, h0 * f_out0), c0),
        pl.BlockSpec((h0, f_out0), c0),
        pl.BlockSpec((h0, f_out0), c0),
        pl.BlockSpec((1, f_out0), c0),
        pl.BlockSpec((f_in1, f_out1), c0),
        pl.BlockSpec((1, f_out1), c0),
        pl.BlockSpec((1, f_out1), c0),
        pl.BlockSpec((1, f_out1), c0),
    ]

    def body(adj_r, x_r, e_r, w0x_r, w0e_r, as0_r, ad0_r, b0_r,
             w1_r, as1_r, ad1_r, b1_r, out_r):
        _gat_body(x_r, e_r, adj_r, w0x_r, w0e_r, as0_r, ad0_r, b0_r,
                  w1_r, as1_r, ad1_r, b1_r, out_r)

    out = pl.pallas_call(
        body,
        grid=grid,
        in_specs=specs,
        out_specs=pl.BlockSpec((1, 1, f_out1), lambda b: (b, 0, 0)),
        out_shape=jax.ShapeDtypeStruct((bs, 1, f_out1), jnp.float32),
        compiler_params=pltpu.CompilerParams(
            dimension_semantics=("arbitrary",)),
    )(adj_i8, x, emb, w0x, w0e, asrc0m, adst0m, b0m,
      w1m, asrc1m, adst1m, b1m)
    return out[:, 0, :]


# speedup vs baseline: 2.7103x; 1.0219x over previous
import jax
import jax.numpy as jnp
from jax.experimental import pallas as pl
from jax.experimental.pallas import tpu as pltpu

_NEG = -1e9


def _leaky(x):
    return jnp.maximum(x, 0.2 * x)


def _gat_body(x_ref, e_ref, adj_ref, w0x_ref, w0e_ref, asrc0_ref, adst0_ref,
              b0_ref, w1_ref, asrc1_ref, adst1_ref, b1_ref, out_ref):
    n = x_ref.shape[1]
    heads, fo = asrc0_ref.shape

    xb = x_ref[0]
    eb = e_ref[0]
    h1 = (jnp.dot(xb, w0x_ref[...], preferred_element_type=jnp.float32)
          + jnp.dot(eb, w0e_ref[...], preferred_element_type=jnp.float32))
    mbias = adj_ref[0]

    outs = []
    for h in range(heads):
        hh = h1[:, h * fo:(h + 1) * fo]
        av = asrc0_ref[h:h + 1, :]
        dv = adst0_ref[h:h + 1, :]
        asrc = jax.lax.dot_general(hh, av, (((1,), (1,)), ((), ())),
                                   preferred_element_type=jnp.float32)
        adst = jax.lax.dot_general(dv, hh, (((1,), (1,)), ((), ())),
                                   preferred_element_type=jnp.float32)
        lg = _leaky(asrc + adst) + mbias
        m = jnp.max(lg, axis=1, keepdims=True)
        ex = jnp.exp(lg - m)
        s = jnp.sum(ex, axis=1, keepdims=True)
        oh = (jnp.dot(ex, hh, preferred_element_type=jnp.float32) / s
              + b0_ref[...])
        outs.append(jnp.where(oh > 0, oh, jnp.exp(oh) - 1.0))

    o1 = jnp.concatenate(outs, axis=1)
    h2 = jnp.dot(o1, w1_ref[...], preferred_element_type=jnp.float32)

    hlast = h2[n - 1:n, :]
    s2 = jax.lax.dot_general(hlast, asrc1_ref[...], (((1,), (1,)), ((), ())),
                             preferred_element_type=jnp.float32)
    d2 = jax.lax.dot_general(adst1_ref[...], h2, (((1,), (1,)), ((), ())),
                             preferred_element_type=jnp.float32)
    lg2 = _leaky(s2 + d2) + mbias[n - 1:n, :]
    m2 = jnp.max(lg2, axis=1, keepdims=True)
    e2 = jnp.exp(lg2 - m2)
    o2 = (jnp.dot(e2, h2, preferred_element_type=jnp.float32)
          / jnp.sum(e2, axis=1, keepdims=True) + b1_ref[...])

    mm = jnp.max(o2, axis=1, keepdims=True)
    z = o2 - mm
    lse = jnp.log(jnp.sum(jnp.exp(z), axis=1, keepdims=True))
    out_ref[0] = z - lse


def kernel(adj, x, normalized_embedding, w0, a_src0, a_dst0, b0,
           w1, a_src1, a_dst1, b1):
    bs, n = adj.shape[:2]
    f_x = x.shape[2]
    f_emb = normalized_embedding.shape[2]
    h0, f_in0, f_out0 = w0.shape
    f_in1, f_out1 = w1.shape[1], w1.shape[2]

    emb = normalized_embedding.astype(jnp.float32)
    w0r = jnp.transpose(w0, (1, 0, 2)).reshape(f_in0, h0 * f_out0)
    w0x, w0e = w0r[:f_x], w0r[f_x:]
    asrc0m = a_src0[:, :, 0]
    adst0m = a_dst0[:, :, 0]
    w1m = w1[0]
    asrc1m = a_src1[0].reshape(1, f_out1)
    adst1m = a_dst1[0].reshape(1, f_out1)
    b0m = b0.reshape(1, f_out0)
    b1m = b1.reshape(1, f_out1)
    mbias = jnp.where(adj, jnp.float32(0), jnp.float32(-1e9))

    grid = (bs,)
    c0 = lambda b: (0, 0)
    specs = [
        pl.BlockSpec((1, n, n), lambda b: (b, 0, 0)),
        pl.BlockSpec((1, n, f_x), lambda b: (b, 0, 0)),
        pl.BlockSpec((1, n, f_emb), lambda b: (b, 0, 0)),
        pl.BlockSpec((f_x, h0 * f_out0), c0),
        pl.BlockSpec((f_emb, h0 * f_out0), c0),
        pl.BlockSpec((h0, f_out0), c0),
        pl.BlockSpec((h0, f_out0), c0),
        pl.BlockSpec((1, f_out0), c0),
        pl.BlockSpec((f_in1, f_out1), c0),
        pl.BlockSpec((1, f_out1), c0),
        pl.BlockSpec((1, f_out1), c0),
        pl.BlockSpec((1, f_out1), c0),
    ]

    def body(adj_r, x_r, e_r, w0x_r, w0e_r, as0_r, ad0_r, b0_r,
             w1_r, as1_r, ad1_r, b1_r, out_r):
        _gat_body(x_r, e_r, adj_r, w0x_r, w0e_r, as0_r, ad0_r, b0_r,
                  w1_r, as1_r, ad1_r, b1_r, out_r)

    out = pl.pallas_call(
        body,
        grid=grid,
        in_specs=specs,
        out_specs=pl.BlockSpec((1, 1, f_out1), lambda b: (b, 0, 0)),
        out_shape=jax.ShapeDtypeStruct((bs, 1, f_out1), jnp.float32),
        compiler_params=pltpu.CompilerParams(
            dimension_semantics=("parallel",)),
    )(mbias, x, emb, w0x, w0e, asrc0m, adst0m, b0m,
      w1m, asrc1m, adst1m, b1m)
    return out[:, 0, :]


# speedup vs baseline: 2.8022x; 1.0339x over previous
import jax
import jax.numpy as jnp
from jax.experimental import pallas as pl
from jax.experimental.pallas import tpu as pltpu

_NEG = -1e9


def _leaky(x):
    return jnp.maximum(x, 0.2 * x)


def _gat_body(x_ref, e_ref, adj_ref, w0x_ref, w0e_ref, asrc0_ref, adst0_ref,
              b0_ref, w1_ref, asrc1_ref, adst1_ref, b1_ref, out_ref):
    n = x_ref.shape[1]
    heads, fo = asrc0_ref.shape

    xb = x_ref[0]
    eb = e_ref[0]
    h1 = (jnp.dot(xb, w0x_ref[...], preferred_element_type=jnp.float32)
          + jnp.dot(eb, w0e_ref[...], preferred_element_type=jnp.float32))
    h1b = h1.astype(jnp.bfloat16)
    mbias = adj_ref[0]

    outs = []
    for h in range(heads):
        hh = h1[:, h * fo:(h + 1) * fo]
        hhb = h1b[:, h * fo:(h + 1) * fo]
        av = asrc0_ref[h:h + 1, :]
        dv = adst0_ref[h:h + 1, :]
        asrc = jax.lax.dot_general(hh, av, (((1,), (1,)), ((), ())),
                                   preferred_element_type=jnp.float32)
        adst = jax.lax.dot_general(dv, hh, (((1,), (1,)), ((), ())),
                                   preferred_element_type=jnp.float32)
        lg = _leaky(asrc + adst) + mbias
        m = jnp.max(lg, axis=1, keepdims=True)
        ex = jnp.exp(lg - m)
        s = jnp.sum(ex, axis=1, keepdims=True)
        exb = ex.astype(jnp.bfloat16)
        oh = (jnp.dot(exb, hhb, preferred_element_type=jnp.float32) / s
              + b0_ref[...])
        outs.append(jnp.where(oh > 0, oh, jnp.exp(oh) - 1.0))

    o1 = jnp.concatenate(outs, axis=1).astype(jnp.bfloat16)
    h2 = jnp.dot(o1, w1_ref[...], preferred_element_type=jnp.float32)

    hlast = h2[n - 1:n, :]
    s2 = jax.lax.dot_general(hlast, asrc1_ref[...], (((1,), (1,)), ((), ())),
                             preferred_element_type=jnp.float32)
    d2 = jax.lax.dot_general(adst1_ref[...], h2, (((1,), (1,)), ((), ())),
                             preferred_element_type=jnp.float32)
    lg2 = _leaky(s2 + d2) + mbias[n - 1:n, :]
    m2 = jnp.max(lg2, axis=1, keepdims=True)
    e2 = jnp.exp(lg2 - m2)
    o2 = (jnp.dot(e2, h2, preferred_element_type=jnp.float32)
          / jnp.sum(e2, axis=1, keepdims=True) + b1_ref[...])

    mm = jnp.max(o2, axis=1, keepdims=True)
    z = o2 - mm
    lse = jnp.log(jnp.sum(jnp.exp(z), axis=1, keepdims=True))
    out_ref[0] = z - lse


def kernel(adj, x, normalized_embedding, w0, a_src0, a_dst0, b0,
           w1, a_src1, a_dst1, b1):
    bs, n = adj.shape[:2]
    f_x = x.shape[2]
    f_emb = normalized_embedding.shape[2]
    h0, f_in0, f_out0 = w0.shape
    f_in1, f_out1 = w1.shape[1], w1.shape[2]

    emb = normalized_embedding.astype(jnp.bfloat16)
    xb16 = x.astype(jnp.bfloat16)
    w0r = jnp.transpose(w0, (1, 0, 2)).reshape(f_in0, h0 * f_out0)
    w0x = w0r[:f_x].astype(jnp.bfloat16)
    w0e = w0r[f_x:].astype(jnp.bfloat16)
    asrc0m = a_src0[:, :, 0]
    adst0m = a_dst0[:, :, 0]
    w1m = w1[0].astype(jnp.bfloat16)
    asrc1m = a_src1[0].reshape(1, f_out1)
    adst1m = a_dst1[0].reshape(1, f_out1)
    b0m = b0.reshape(1, f_out0)
    b1m = b1.reshape(1, f_out1)
    mbias = jnp.where(adj, jnp.float32(0), jnp.float32(-1e9))

    grid = (bs,)
    c0 = lambda b: (0, 0)
    specs = [
        pl.BlockSpec((1, n, n), lambda b: (b, 0, 0)),
        pl.BlockSpec((1, n, f_x), lambda b: (b, 0, 0)),
        pl.BlockSpec((1, n, f_emb), lambda b: (b, 0, 0)),
        pl.BlockSpec((f_x, h0 * f_out0), c0),
        pl.BlockSpec((f_emb, h0 * f_out0), c0),
        pl.BlockSpec((h0, f_out0), c0),
        pl.BlockSpec((h0, f_out0), c0),
        pl.BlockSpec((1, f_out0), c0),
        pl.BlockSpec((f_in1, f_out1), c0),
        pl.BlockSpec((1, f_out1), c0),
        pl.BlockSpec((1, f_out1), c0),
        pl.BlockSpec((1, f_out1), c0),
    ]

    def body(adj_r, x_r, e_r, w0x_r, w0e_r, as0_r, ad0_r, b0_r,
             w1_r, as1_r, ad1_r, b1_r, out_r):
        _gat_body(x_r, e_r, adj_r, w0x_r, w0e_r, as0_r, ad0_r, b0_r,
                  w1_r, as1_r, ad1_r, b1_r, out_r)

    out = pl.pallas_call(
        body,
        grid=grid,
        in_specs=specs,
        out_specs=pl.BlockSpec((1, 1, f_out1), lambda b: (b, 0, 0)),
        out_shape=jax.ShapeDtypeStruct((bs, 1, f_out1), jnp.float32),
        compiler_params=pltpu.CompilerParams(
            dimension_semantics=("parallel",)),
    )(mbias, xb16, emb, w0x, w0e, asrc0m, adst0m, b0m,
      w1m, asrc1m, adst1m, b1m)
    return out[:, 0, :]


# speedup vs baseline: 3.0101x; 1.0742x over previous
import jax
import jax.numpy as jnp
from jax.experimental import pallas as pl
from jax.experimental.pallas import tpu as pltpu

_NEG = -1e9


def _leaky(x):
    return jnp.maximum(x, 0.2 * x)


def _gat_body(x_ref, e_ref, adj_ref, w0x_ref, w0e_ref, asrc0_ref, adst0_ref,
              b0_ref, w1_ref, asrc1_ref, adst1_ref, b1_ref, out_ref):
    n = x_ref.shape[1]
    heads, fo = asrc0_ref.shape

    xb = x_ref[0]
    eb = e_ref[0]
    h1 = (jnp.dot(xb, w0x_ref[...], preferred_element_type=jnp.float32)
          + jnp.dot(eb, w0e_ref[...], preferred_element_type=jnp.float32))
    h1b = h1.astype(jnp.bfloat16)
    mbias = (adj_ref[0].astype(jnp.float32) - 1.0) * 1e9

    outs = []
    for h in range(heads):
        hh = h1[:, h * fo:(h + 1) * fo]
        hhb = h1b[:, h * fo:(h + 1) * fo]
        av = asrc0_ref[h:h + 1, :]
        dv = adst0_ref[h:h + 1, :]
        asrc = jax.lax.dot_general(hh, av, (((1,), (1,)), ((), ())),
                                   preferred_element_type=jnp.float32)
        adst = jax.lax.dot_general(dv, hh, (((1,), (1,)), ((), ())),
                                   preferred_element_type=jnp.float32)
        lg = _leaky(asrc + adst) + mbias
        m = jnp.max(lg, axis=1, keepdims=True)
        ex = jnp.exp2(lg - m)
        s = jnp.sum(ex, axis=1, keepdims=True)
        exb = ex.astype(jnp.bfloat16)
        oh = (jnp.dot(exb, hhb, preferred_element_type=jnp.float32) / s
              + b0_ref[...])
        outs.append(jnp.where(oh > 0, oh, jnp.exp(oh) - 1.0))

    o1 = jnp.concatenate(outs, axis=1).astype(jnp.bfloat16)
    h2 = jnp.dot(o1, w1_ref[...], preferred_element_type=jnp.float32)

    hlast = h2[n - 1:n, :]
    s2 = jax.lax.dot_general(hlast, asrc1_ref[...], (((1,), (1,)), ((), ())),
                             preferred_element_type=jnp.float32)
    d2 = jax.lax.dot_general(adst1_ref[...], h2, (((1,), (1,)), ((), ())),
                             preferred_element_type=jnp.float32)
    lg2 = _leaky(s2 + d2) + mbias[n - 1:n, :]
    m2 = jnp.max(lg2, axis=1, keepdims=True)
    e2 = jnp.exp2(lg2 - m2)
    o2 = (jnp.dot(e2, h2, preferred_element_type=jnp.float32)
          / jnp.sum(e2, axis=1, keepdims=True) + b1_ref[...])

    mm = jnp.max(o2, axis=1, keepdims=True)
    z = o2 - mm
    lse = jnp.log(jnp.sum(jnp.exp(z), axis=1, keepdims=True))
    out_ref[0] = z - lse


def kernel(adj, x, normalized_embedding, w0, a_src0, a_dst0, b0,
           w1, a_src1, a_dst1, b1):
    bs, n = adj.shape[:2]
    f_x = x.shape[2]
    f_emb = normalized_embedding.shape[2]
    h0, f_in0, f_out0 = w0.shape
    f_in1, f_out1 = w1.shape[1], w1.shape[2]

    emb = normalized_embedding.astype(jnp.bfloat16)
    xb16 = x.astype(jnp.bfloat16)
    w0r = jnp.transpose(w0, (1, 0, 2)).reshape(f_in0, h0 * f_out0)
    w0x = w0r[:f_x].astype(jnp.bfloat16)
    w0e = w0r[f_x:].astype(jnp.bfloat16)
    log2e = jnp.float32(1.4426950408889634)
    asrc0m = a_src0[:, :, 0] * log2e
    adst0m = a_dst0[:, :, 0] * log2e
    w1m = w1[0].astype(jnp.bfloat16)
    asrc1m = a_src1[0].reshape(1, f_out1) * log2e
    adst1m = a_dst1[0].reshape(1, f_out1) * log2e
    b0m = b0.reshape(1, f_out0)
    b1m = b1.reshape(1, f_out1)
    adj_i8 = adj.astype(jnp.int8)

    grid = (bs,)
    c0 = lambda b: (0, 0)
    specs = [
        pl.BlockSpec((1, n, n), lambda b: (b, 0, 0)),
        pl.BlockSpec((1, n, f_x), lambda b: (b, 0, 0)),
        pl.BlockSpec((1, n, f_emb), lambda b: (b, 0, 0)),
        pl.BlockSpec((f_x, h0 * f_out0), c0),
        pl.BlockSpec((f_emb, h0 * f_out0), c0),
        pl.BlockSpec((h0, f_out0), c0),
        pl.BlockSpec((h0, f_out0), c0),
        pl.BlockSpec((1, f_out0), c0),
        pl.BlockSpec((f_in1, f_out1), c0),
        pl.BlockSpec((1, f_out1), c0),
        pl.BlockSpec((1, f_out1), c0),
        pl.BlockSpec((1, f_out1), c0),
    ]

    def body(adj_r, x_r, e_r, w0x_r, w0e_r, as0_r, ad0_r, b0_r,
             w1_r, as1_r, ad1_r, b1_r, out_r):
        _gat_body(x_r, e_r, adj_r, w0x_r, w0e_r, as0_r, ad0_r, b0_r,
                  w1_r, as1_r, ad1_r, b1_r, out_r)

    out = pl.pallas_call(
        body,
        grid=grid,
        in_specs=specs,
        out_specs=pl.BlockSpec((1, 1, f_out1), lambda b: (b, 0, 0)),
        out_shape=jax.ShapeDtypeStruct((bs, 1, f_out1), jnp.float32),
        compiler_params=pltpu.CompilerParams(
            dimension_semantics=("parallel",)),
    )(adj_i8, xb16, emb, w0x, w0e, asrc0m, adst0m, b0m,
      w1m, asrc1m, adst1m, b1m)
    return out[:, 0, :]
